# Initial kernel scaffold; baseline (speedup 1.0000x reference)
import jax
import jax.numpy as jnp
from jax.experimental import pallas as pl
from jax.experimental.pallas import tpu as pltpu

_N_NODES = 10000
_N_GRAPHS = 64
_D_NODE = 128
_D_GLOBAL = 8


def _tc_body(x_ref, xr_ref, batch_ref, u_ref, ur_ref, wn_ref, wu_ref, b_ref,
             out_ref):
    d = x_ref[...] - xr_ref[...]
    s = jnp.sum(d * wn_ref[...], axis=1, keepdims=True)
    gids = jax.lax.broadcasted_iota(jnp.int32, (_N_NODES, _N_GRAPHS), 1)
    onehot = (batch_ref[...] == gids).astype(jnp.float32)
    sums = jax.lax.dot_general(onehot, s, (((0,), (0,)), ((), ())),
                               preferred_element_type=jnp.float32)
    counts = jnp.sum(onehot, axis=0)[:, None]
    uterm = jnp.sum((u_ref[...] - ur_ref[...]) * wu_ref[...], axis=1,
                    keepdims=True)
    out_ref[...] = sums / jnp.maximum(counts, 1.0) + uterm + b_ref[...]


def kernel(x, x_reactant, edge_index, edge_index_reactant, edge_attr,
           edge_attr_reactant, u, u_reactant, batch, W, b):
    del edge_index, edge_index_reactant, edge_attr, edge_attr_reactant
    u = u.reshape(-1, _D_GLOBAL)
    u_reactant = u_reactant.reshape(-1, _D_GLOBAL)
    wn = W[_D_GLOBAL:_D_GLOBAL + _D_NODE].reshape(1, _D_NODE)
    wu = W[:_D_GLOBAL].reshape(1, _D_GLOBAL)
    b2 = b.reshape(1, 1)
    batch2 = batch.astype(jnp.int32).reshape(_N_NODES, 1)

    out = pl.pallas_call(
        _tc_body,
        out_shape=jax.ShapeDtypeStruct((_N_GRAPHS, 1), jnp.float32),
    )(x, x_reactant, batch2, u, u_reactant, wn, wu, b2)
    return out


# speedup vs baseline: 115.3459x; 115.3459x over previous
import jax
import jax.numpy as jnp
from jax.experimental import pallas as pl
from jax.experimental.pallas import tpu as pltpu

_N_NODES = 10000
_N_GRAPHS = 64
_D_NODE = 128
_D_GLOBAL = 8


def _tc_body(x_ref, xr_ref, batch_ref, u_ref, ur_ref, wn_ref, wu_ref, b_ref,
             out_ref):
    d = x_ref[...] - xr_ref[...]
    s = jnp.sum(d * wn_ref[...], axis=1, keepdims=True)
    gids = jax.lax.broadcasted_iota(jnp.int32, (_N_NODES, _N_GRAPHS), 1)
    onehot = (batch_ref[...] == gids).astype(jnp.float32)
    sums = jax.lax.dot_general(onehot, s, (((0,), (0,)), ((), ())),
                               preferred_element_type=jnp.float32,
                               precision=jax.lax.Precision.HIGHEST)
    counts = jnp.sum(onehot, axis=0)[:, None]
    uterm = jnp.sum((u_ref[...] - ur_ref[...]) * wu_ref[...], axis=1,
                    keepdims=True)
    out_ref[...] = sums / jnp.maximum(counts, 1.0) + uterm + b_ref[...]


def kernel(x, x_reactant, edge_index, edge_index_reactant, edge_attr,
           edge_attr_reactant, u, u_reactant, batch, W, b):
    del edge_index, edge_index_reactant, edge_attr, edge_attr_reactant
    u = u.reshape(-1, _D_GLOBAL)
    u_reactant = u_reactant.reshape(-1, _D_GLOBAL)
    wn = W[_D_GLOBAL:_D_GLOBAL + _D_NODE].reshape(1, _D_NODE)
    wu = W[:_D_GLOBAL].reshape(1, _D_GLOBAL)
    b2 = b.reshape(1, 1)
    batch2 = batch.astype(jnp.int32).reshape(_N_NODES, 1)

    out = pl.pallas_call(
        _tc_body,
        out_shape=jax.ShapeDtypeStruct((_N_GRAPHS, 1), jnp.float32),
    )(x, x_reactant, batch2, u, u_reactant, wn, wu, b2)
    return out
